# baseline (speedup 1.0000x reference)
import jax
import jax.numpy as jnp
from jax import lax
from jax.experimental import pallas as pl
from jax.experimental.pallas import tpu as pltpu
from jax.experimental.pallas import tpu_sc as plsc

_MAX_WEIGHT_NORM = 0.01
_L = 16
_NT = 16


def _lane_total(t):
    idx = lax.iota(jnp.int32, _L)
    dnums = lax.GatherDimensionNumbers(
        offset_dims=(), collapsed_slice_dims=(0,), start_index_map=(0,))
    for sh in (8, 4, 2, 1):
        perm = lax.bitwise_xor(idx, jnp.int32(sh))
        peer = lax.gather(t, perm[:, None], dnums, (1,),
                          mode=lax.GatherScatterMode.PROMISE_IN_BOUNDS)
        t = t + peer
    return t


def _newton_rsqrt(a):
    i = lax.bitcast_convert_type(a, jnp.int32)
    i = jnp.int32(0x5F3759DF) - lax.shift_right_logical(i, 1)
    y = lax.bitcast_convert_type(i, jnp.float32)
    for _ in range(4):
        y = y * (1.5 - 0.5 * (a * y) * y)
    return y


def _make_sc_body(chunk, iters):
    def body(special_hbm, common_hbm, out_hbm, sp_v, cm_v, acc_v, part_v, shared, sem):
        sid = lax.axis_index("s")
        base = sid * chunk
        pltpu.sync_copy(special_hbm.at[pl.ds(base, chunk)], sp_v)
        cm_dma = pltpu.async_copy(common_hbm.at[pl.ds(base, chunk)], cm_v, sem)

        def ssq_step(i, acc):
            v = sp_v[pl.ds(i * _L, _L)]
            return acc + v * v

        acc = lax.fori_loop(0, iters, ssq_step, jnp.zeros((_L,), jnp.float32))
        acc_v[...] = acc
        pltpu.sync_copy(acc_v, shared.at[sid])
        plsc.subcore_barrier()
        pltpu.sync_copy(shared, part_v)
        tot = part_v[0, :]
        for r in range(1, _NT):
            tot = tot + part_v[r, :]
        ssq_v = _lane_total(tot)
        scale = jnp.minimum(jnp.float32(1.0), _MAX_WEIGHT_NORM * _newton_rsqrt(ssq_v))

        cm_dma.wait()

        def out_step(i, carry):
            sl = pl.ds(i * _L, _L)
            sp_v[sl] = scale * sp_v[sl] + cm_v[sl]
            return carry

        lax.fori_loop(0, iters, out_step, 0)
        pltpu.sync_copy(sp_v, out_hbm.at[pl.ds(base, chunk)])

    return body


def kernel(x, lens, common_latent, special_latent):
    del x, lens
    rows, cols = special_latent.shape
    n = rows * cols
    chunk = n // _NT
    sp = special_latent.reshape(n)
    cm = common_latent.reshape(n)
    mesh = plsc.VectorSubcoreMesh(core_axis_name="c", subcore_axis_name="s", num_cores=1)
    out = pl.kernel(
        _make_sc_body(chunk, chunk // _L),
        out_type=jax.ShapeDtypeStruct((n,), jnp.float32),
        mesh=mesh,
        scratch_types=[
            pltpu.VMEM((chunk,), jnp.float32),
            pltpu.VMEM((chunk,), jnp.float32),
            pltpu.VMEM((_L,), jnp.float32),
            pltpu.VMEM((_NT, _L), jnp.float32),
            pltpu.VMEM_SHARED((_NT, _L), jnp.float32),
            pltpu.SemaphoreType.DMA,
        ],
    )(sp, cm)
    return out.reshape(rows, cols)


# speedup vs baseline: 3.7408x; 3.7408x over previous
import jax
import jax.numpy as jnp
from jax.experimental import pallas as pl
from jax.experimental.pallas import tpu as pltpu

_MAX_WEIGHT_NORM = 0.01
_NCH = 4


def _make_body(rows, cols):
    rch = rows // _NCH

    def body(sp_hbm, cm_hbm, out_hbm, sp_v, cm_v, o_v, sp_sems, cm_sems, out_sems):
        def chunk(ref, i):
            return ref.at[pl.ds(i * rch, rch)]

        sp_dmas = [
            pltpu.make_async_copy(chunk(sp_hbm, i), chunk(sp_v, i), sp_sems.at[i])
            for i in range(_NCH)
        ]
        cm_dmas = [
            pltpu.make_async_copy(chunk(cm_hbm, i), chunk(cm_v, i), cm_sems.at[i])
            for i in range(_NCH)
        ]
        for d in sp_dmas:
            d.start()
        for d in cm_dmas:
            d.start()

        ssq = jnp.float32(0.0)
        for i in range(_NCH):
            sp_dmas[i].wait()
            s = sp_v[pl.ds(i * rch, rch), :]
            ssq = ssq + jnp.sum(s * s)

        norm = jnp.sqrt(ssq)
        scale = jnp.where(norm > _MAX_WEIGHT_NORM, _MAX_WEIGHT_NORM / norm, 1.0)

        out_dmas = [
            pltpu.make_async_copy(chunk(o_v, i), chunk(out_hbm, i), out_sems.at[i])
            for i in range(_NCH)
        ]
        for i in range(_NCH):
            cm_dmas[i].wait()
            sl = (pl.ds(i * rch, rch), slice(None))
            o_v[sl] = sp_v[sl] * scale + cm_v[sl]
            out_dmas[i].start()
        for d in out_dmas:
            d.wait()

    return body


def kernel(x, lens, common_latent, special_latent):
    del x, lens
    rows, cols = special_latent.shape
    return pl.pallas_call(
        _make_body(rows, cols),
        in_specs=[
            pl.BlockSpec(memory_space=pltpu.HBM),
            pl.BlockSpec(memory_space=pltpu.HBM),
        ],
        out_specs=pl.BlockSpec(memory_space=pltpu.HBM),
        out_shape=jax.ShapeDtypeStruct((rows, cols), special_latent.dtype),
        scratch_shapes=[
            pltpu.VMEM((rows, cols), jnp.float32),
            pltpu.VMEM((rows, cols), jnp.float32),
            pltpu.VMEM((rows, cols), jnp.float32),
            pltpu.SemaphoreType.DMA((_NCH,)),
            pltpu.SemaphoreType.DMA((_NCH,)),
            pltpu.SemaphoreType.DMA((_NCH,)),
        ],
    )(special_latent, common_latent)


# speedup vs baseline: 3.8732x; 1.0354x over previous
import jax
import jax.numpy as jnp
from jax.experimental import pallas as pl
from jax.experimental.pallas import tpu as pltpu

_MAX_WEIGHT_NORM = 0.01
_NCH = 8


def _make_body(rows, cols):
    rch = rows // _NCH

    def body(sp_hbm, cm_hbm, out_hbm, sp_v, cm_v, o_v, sp_sems, cm_sems, out_sems):
        def chunk(ref, i):
            return ref.at[pl.ds(i * rch, rch)]

        sp_dmas = [
            pltpu.make_async_copy(chunk(sp_hbm, i), chunk(sp_v, i), sp_sems.at[i])
            for i in range(_NCH)
        ]
        cm_dmas = [
            pltpu.make_async_copy(chunk(cm_hbm, i), chunk(cm_v, i), cm_sems.at[i])
            for i in range(_NCH)
        ]
        for d in sp_dmas:
            d.start()
        for d in cm_dmas:
            d.start()

        acc = jnp.zeros((rch, cols), jnp.float32)
        for i in range(_NCH):
            sp_dmas[i].wait()
            s = sp_v[pl.ds(i * rch, rch), :]
            acc = acc + s * s
        ssq = jnp.sum(acc)

        norm = jnp.sqrt(ssq)
        scale = jnp.where(norm > _MAX_WEIGHT_NORM, _MAX_WEIGHT_NORM / norm, 1.0)

        out_dmas = [
            pltpu.make_async_copy(chunk(o_v, i), chunk(out_hbm, i), out_sems.at[i])
            for i in range(_NCH)
        ]
        for i in range(_NCH):
            cm_dmas[i].wait()
            sl = (pl.ds(i * rch, rch), slice(None))
            o_v[sl] = sp_v[sl] * scale + cm_v[sl]
            out_dmas[i].start()
        for d in out_dmas:
            d.wait()

    return body


def kernel(x, lens, common_latent, special_latent):
    del x, lens
    rows, cols = special_latent.shape
    return pl.pallas_call(
        _make_body(rows, cols),
        in_specs=[
            pl.BlockSpec(memory_space=pltpu.HBM),
            pl.BlockSpec(memory_space=pltpu.HBM),
        ],
        out_specs=pl.BlockSpec(memory_space=pltpu.HBM),
        out_shape=jax.ShapeDtypeStruct((rows, cols), special_latent.dtype),
        scratch_shapes=[
            pltpu.VMEM((rows, cols), jnp.float32),
            pltpu.VMEM((rows, cols), jnp.float32),
            pltpu.VMEM((rows, cols), jnp.float32),
            pltpu.SemaphoreType.DMA((_NCH,)),
            pltpu.SemaphoreType.DMA((_NCH,)),
            pltpu.SemaphoreType.DMA((_NCH,)),
        ],
        compiler_params=pltpu.CompilerParams(
            disable_bounds_checks=True,
            disable_semaphore_checks=True,
            skip_device_barrier=True,
        ),
    )(special_latent, common_latent)
